# baseline (speedup 1.0000x reference)
import functools

import jax
import jax.numpy as jnp
from jax import lax
from jax.experimental import pallas as pl
from jax.experimental.pallas import tpu as pltpu
from jax.experimental.pallas import tpu_sc as plsc

N_NODES = 10000
N_EDGES = 320000

NC = 2
NS = 16
NW = NC * NS
NROWS = 10240
STRIPE = NROWS // NS
TRASH = N_NODES

_PLAN = {
    128: dict(ch=64, n0=320, n1=0),
    64: dict(ch=128, n0=130, n1=30),
    32: dict(ch=128, n0=120, n1=40),
    16: dict(ch=128, n0=96, n1=64),
}


def _make_agg(D):
  plan = _PLAN[D]
  ch, n0, n1 = plan["ch"], plan["n0"], plan["n1"]
  nc_out = NC if n1 > 0 else 1
  seg = 80 if D == 128 else max(n0, n1)
  mesh = plsc.VectorSubcoreMesh(
      core_axis_name="c", subcore_axis_name="s", num_cores=NC, num_subcores=NS)

  @functools.partial(
      pl.kernel,
      out_type=jax.ShapeDtypeStruct((nc_out, NROWS, D), jnp.float32),
      mesh=mesh,
      compiler_params=pltpu.CompilerParams(use_tc_tiling_on_sc=False),
      scratch_types=[
          pltpu.VMEM((seg, ch), jnp.int32),
          pltpu.VMEM((seg, ch), jnp.int32),
          pltpu.VMEM((ch, D), jnp.float32),
          pltpu.VMEM((ch, D), jnp.float32),
          pltpu.VMEM_SHARED((NROWS, D), jnp.float32),
          pltpu.SemaphoreType.DMA,
          pltpu.SemaphoreType.DMA,
      ],
  )
  def agg(u_hbm, src_hbm, dst_hbm, zero_hbm, out_hbm,
          src_v, dst_v, msg0, msg1, acc, sem0, sem1):
    c = lax.axis_index("c")
    s = lax.axis_index("s")
    base = s * STRIPE

    def zero_acc():
      pltpu.sync_copy(zero_hbm, msg1)
      for k in range(STRIPE // ch):
        pltpu.sync_copy(msg1, acc.at[pl.ds(base + k * ch, ch)])

    def run_core(start, ntot):
      for g in range(-(-ntot // seg)):
        cnt = min(seg, ntot - g * seg)
        pltpu.sync_copy(src_hbm.at[pl.ds(start + g * seg, seg)], src_v)
        pltpu.sync_copy(dst_hbm.at[pl.ds(start + g * seg, seg)], dst_v)
        pltpu.async_copy(u_hbm.at[src_v.at[0]], msg0, sem0)
        npair = cnt // 2

        def body(i, carry):
          a = 2 * i
          pltpu.make_async_copy(u_hbm.at[src_v.at[a]], msg0, sem0).wait()
          pltpu.async_copy(u_hbm.at[src_v.at[a + 1]], msg1, sem1)
          pltpu.sync_copy(msg0, acc.at[dst_v.at[a]], add=True)
          pltpu.make_async_copy(u_hbm.at[src_v.at[a + 1]], msg1, sem1).wait()

          @pl.when(i < npair - 1)
          def _():
            pltpu.async_copy(u_hbm.at[src_v.at[a + 2]], msg0, sem0)

          pltpu.sync_copy(msg1, acc.at[dst_v.at[a + 1]], add=True)
          return carry

        lax.fori_loop(0, npair, body, 0)

    def writeback():
      for k in range(STRIPE // ch):
        pltpu.sync_copy(acc.at[pl.ds(base + k * ch, ch)], msg0)
        pltpu.sync_copy(msg0, out_hbm.at[c, pl.ds(base + k * ch, ch)])

    if n1 > 0:
      zero_acc()
      plsc.subcore_barrier()

      @pl.when(c == 0)
      def _():
        run_core(s * n0, n0)

      @pl.when(c == 1)
      def _():
        run_core(NS * n0 + s * n1, n1)

      plsc.subcore_barrier()
      writeback()
    else:
      @pl.when(c == 0)
      def _():
        zero_acc()

      plsc.subcore_barrier()

      @pl.when(c == 0)
      def _():
        run_core(s * n0, n0)

      plsc.subcore_barrier()

      @pl.when(c == 0)
      def _():
        writeback()

  return agg


_agg_cache = {}


def _agg(u, src_flat, dst_flat, D):
  if D not in _agg_cache:
    _agg_cache[D] = _make_agg(D)
  plan = _PLAN[D]
  ch, n0, n1 = plan["ch"], plan["n0"], plan["n1"]
  nchunk = NS * (n0 + n1)
  ne = nchunk * ch
  pads = jnp.zeros(((nchunk + n0) * ch - N_EDGES,), jnp.int32)
  padd = jnp.full(((nchunk + n0) * ch - N_EDGES,), TRASH, jnp.int32)
  src2 = jnp.concatenate([src_flat, pads]).reshape(nchunk + n0, ch)
  dst2 = jnp.concatenate([dst_flat, padd]).reshape(nchunk + n0, ch)
  zero = jnp.zeros((ch, D), jnp.float32)
  del ne
  p = _agg_cache[D](u, src2, dst2, zero)
  p0 = p[0, :N_NODES]
  if p.shape[0] == 2:
    p1 = p[1, :N_NODES]
  else:
    p1 = jnp.zeros_like(p0)
  return p0, p1


R = 1000
GRID = (N_NODES // R,)


def _row_spec(d):
  return pl.BlockSpec((R, d), lambda i: (i, 0))


def _full_spec(shape):
  return pl.BlockSpec(shape, lambda i: tuple(0 for _ in shape))


def _tc_pre_body(d0_ref, d1_ref, x_ref, dinv_ref, u1_ref):
  deg = d0_ref[...] + d1_ref[...] + 1.0
  dv = lax.rsqrt(deg)
  dinv_ref[...] = dv
  u1_ref[...] = dv * x_ref[...]


def _tc_pre(d0, d1, x):
  return pl.pallas_call(
      _tc_pre_body,
      grid=GRID,
      in_specs=[_row_spec(1), _row_spec(1), _row_spec(128)],
      out_specs=[_row_spec(1), _row_spec(128)],
      out_shape=[
          jax.ShapeDtypeStruct((N_NODES, 1), jnp.float32),
          jax.ShapeDtypeStruct((N_NODES, 128), jnp.float32),
      ],
  )(d0, d1, x)


def _tc1_body(p0, p1, u, dinv, W1, b1, W2, o):
  dv = dinv[...]
  a = dv * (p0[...] + p1[...] + u[...])
  h = jnp.maximum(jnp.dot(a, W1[...], preferred_element_type=jnp.float32)
                  + b1[...], 0.0)
  o[...] = dv * jnp.dot(h, W2[...], preferred_element_type=jnp.float32)


def _tc1(p0, p1, u, dinv, W1, b1, W2):
  return pl.pallas_call(
      _tc1_body,
      grid=GRID,
      in_specs=[_row_spec(128), _row_spec(128), _row_spec(128), _row_spec(1),
                _full_spec((128, 256)), _full_spec((1, 256)),
                _full_spec((256, 128))],
      out_specs=_row_spec(128),
      out_shape=jax.ShapeDtypeStruct((N_NODES, 128), jnp.float32),
  )(p0, p1, u, dinv, W1, b1, W2)


def _tc_mid_body(p0, p1, u, dinv, b, Wn, o):
  dv = dinv[...]
  h = jnp.maximum(dv * (p0[...] + p1[...] + u[...]) + b[...], 0.0)
  o[...] = dv * jnp.dot(h, Wn[...], preferred_element_type=jnp.float32)


def _tc_mid(p0, p1, u, dinv, b, Wn):
  d = u.shape[1]
  dn = Wn.shape[1]
  return pl.pallas_call(
      _tc_mid_body,
      grid=GRID,
      in_specs=[_row_spec(d), _row_spec(d), _row_spec(d), _row_spec(1),
                _full_spec((1, d)), _full_spec((d, dn))],
      out_specs=_row_spec(dn),
      out_shape=jax.ShapeDtypeStruct((N_NODES, dn), jnp.float32),
  )(p0, p1, u, dinv, b, Wn)


def _tc_h4_body(p0, p1, u, dinv, b, o):
  dv = dinv[...]
  h = jnp.maximum(dv * (p0[...] + p1[...] + u[...]) + b[...], 0.0)
  o[...] = dv * h


def _tc_h4(p0, p1, u, dinv, b):
  d = u.shape[1]
  return pl.pallas_call(
      _tc_h4_body,
      grid=GRID,
      in_specs=[_row_spec(d), _row_spec(d), _row_spec(d), _row_spec(1),
                _full_spec((1, d))],
      out_specs=_row_spec(d),
      out_shape=jax.ShapeDtypeStruct((N_NODES, d), jnp.float32),
  )(p0, p1, u, dinv, b)


def _tc_fin_body(p0, p1, u, dinv, Wm, bm, Wl, bl, mu, ls):
  a = dinv[...] * (p0[...] + p1[...] + u[...])
  mu[...] = jnp.dot(a, Wm[...], preferred_element_type=jnp.float32) + bm[...]
  ls[...] = jnp.dot(a, Wl[...], preferred_element_type=jnp.float32) + bl[...]


def _tc_fin(p0, p1, u, dinv, Wm, bm, Wl, bl):
  return pl.pallas_call(
      _tc_fin_body,
      grid=GRID,
      in_specs=[_row_spec(32), _row_spec(32), _row_spec(32), _row_spec(1),
                _full_spec((32, 16)), _full_spec((1, 16)),
                _full_spec((32, 16)), _full_spec((1, 16))],
      out_specs=[_row_spec(16), _row_spec(16)],
      out_shape=[
          jax.ShapeDtypeStruct((N_NODES, 16), jnp.float32),
          jax.ShapeDtypeStruct((N_NODES, 16), jnp.float32),
      ],
  )(p0, p1, u, dinv, Wm, bm, Wl, bl)


def kernel(x, edge_index, W1, b1, W2, b2, W3, b3, W4, b4,
           W_mu, b_mu, W_logstd, b_logstd):
  src = edge_index[0].astype(jnp.int32)
  dst = edge_index[1].astype(jnp.int32)

  b1r = b1.reshape(1, -1)
  b2r = b2.reshape(1, -1)
  b3r = b3.reshape(1, -1)
  b4r = b4.reshape(1, -1)
  bmr = b_mu.reshape(1, -1)
  blr = b_logstd.reshape(1, -1)

  ones = jnp.ones((N_NODES, 16), jnp.float32)
  g0, g1 = _agg(ones, src, dst, 16)
  dinv, u1 = _tc_pre(g0[:, :1], g1[:, :1], x)

  p0, p1 = _agg(u1, src, dst, 128)
  u2 = _tc1(p0, p1, u1, dinv, W1, b1r, W2)

  p0, p1 = _agg(u2, src, dst, 128)
  u3 = _tc_mid(p0, p1, u2, dinv, b2r, W3)

  p0, p1 = _agg(u3, src, dst, 64)
  u4 = _tc_mid(p0, p1, u3, dinv, b3r, W4)

  p0, p1 = _agg(u4, src, dst, 32)
  u5 = _tc_h4(p0, p1, u4, dinv, b4r)

  p0, p1 = _agg(u5, src, dst, 32)
  mu, logstd = _tc_fin(p0, p1, u5, dinv, W_mu, bmr, W_logstd, blr)
  return (mu, logstd)


# speedup vs baseline: 1.3571x; 1.3571x over previous
import functools

import jax
import jax.numpy as jnp
from jax import lax
from jax.experimental import pallas as pl
from jax.experimental.pallas import tpu as pltpu
from jax.experimental.pallas import tpu_sc as plsc

N_NODES = 10000
N_EDGES = 320000

NC = 2
NS = 16
NW = NC * NS
NROWS = 10240
STRIPE = NROWS // NS
TRASH = N_NODES

_PLAN = {
    128: dict(ch=64, n0=272, n1=48),
    64: dict(ch=128, n0=132, n1=28),
    32: dict(ch=128, n0=120, n1=40),
    16: dict(ch=128, n0=96, n1=64),
}


def _make_agg(D):
  plan = _PLAN[D]
  ch, n0, n1 = plan["ch"], plan["n0"], plan["n1"]
  nc_out = NC if n1 > 0 else 1
  seg = 80 if D == 128 else max(n0, n1)
  mesh = plsc.VectorSubcoreMesh(
      core_axis_name="c", subcore_axis_name="s", num_cores=NC, num_subcores=NS)

  @functools.partial(
      pl.kernel,
      out_type=jax.ShapeDtypeStruct((nc_out, NROWS, D), jnp.float32),
      mesh=mesh,
      compiler_params=pltpu.CompilerParams(use_tc_tiling_on_sc=False),
      scratch_types=[
          pltpu.VMEM((seg, ch), jnp.int32),
          pltpu.VMEM((seg, ch), jnp.int32),
          pltpu.VMEM((ch, D), jnp.float32),
          pltpu.VMEM((ch, D), jnp.float32),
          pltpu.VMEM_SHARED((NROWS, D), jnp.float32),
          pltpu.SemaphoreType.DMA,
          pltpu.SemaphoreType.DMA,
      ],
  )
  def agg(u_hbm, src_hbm, dst_hbm, zero_hbm, out_hbm,
          src_v, dst_v, msg0, msg1, acc, sem0, sem1):
    c = lax.axis_index("c")
    s = lax.axis_index("s")
    base = s * STRIPE

    def zero_acc():
      pltpu.sync_copy(zero_hbm, msg1)
      for k in range(STRIPE // ch):
        pltpu.sync_copy(msg1, acc.at[pl.ds(base + k * ch, ch)])

    def run_core(start, ntot):
      for g in range(-(-ntot // seg)):
        cnt = min(seg, ntot - g * seg)
        pltpu.sync_copy(src_hbm.at[pl.ds(start + g * seg, seg)], src_v)
        pltpu.sync_copy(dst_hbm.at[pl.ds(start + g * seg, seg)], dst_v)
        pltpu.async_copy(u_hbm.at[src_v.at[0]], msg0, sem0)
        npair = cnt // 2

        def body(i, carry):
          a = 2 * i
          pltpu.make_async_copy(u_hbm.at[src_v.at[a]], msg0, sem0).wait()
          pltpu.async_copy(u_hbm.at[src_v.at[a + 1]], msg1, sem1)
          pltpu.sync_copy(msg0, acc.at[dst_v.at[a]], add=True)
          pltpu.make_async_copy(u_hbm.at[src_v.at[a + 1]], msg1, sem1).wait()

          @pl.when(i < npair - 1)
          def _():
            pltpu.async_copy(u_hbm.at[src_v.at[a + 2]], msg0, sem0)

          pltpu.sync_copy(msg1, acc.at[dst_v.at[a + 1]], add=True)
          return carry

        lax.fori_loop(0, npair, body, 0)

    def writeback():
      for k in range(STRIPE // ch):
        pltpu.sync_copy(acc.at[pl.ds(base + k * ch, ch)], msg0)
        pltpu.sync_copy(msg0, out_hbm.at[c, pl.ds(base + k * ch, ch)])

    if n1 > 0:
      zero_acc()
      plsc.subcore_barrier()

      @pl.when(c == 0)
      def _():
        run_core(s * n0, n0)

      @pl.when(c == 1)
      def _():
        run_core(NS * n0 + s * n1, n1)

      plsc.subcore_barrier()
      writeback()
    else:
      @pl.when(c == 0)
      def _():
        zero_acc()

      plsc.subcore_barrier()

      @pl.when(c == 0)
      def _():
        run_core(s * n0, n0)

      plsc.subcore_barrier()

      @pl.when(c == 0)
      def _():
        writeback()

  return agg


_agg_cache = {}


def _agg(u, src_flat, dst_flat, D):
  if D not in _agg_cache:
    _agg_cache[D] = _make_agg(D)
  plan = _PLAN[D]
  ch, n0, n1 = plan["ch"], plan["n0"], plan["n1"]
  nchunk = NS * (n0 + n1)
  ne = nchunk * ch
  pads = jnp.zeros(((nchunk + n0) * ch - N_EDGES,), jnp.int32)
  padd = jnp.full(((nchunk + n0) * ch - N_EDGES,), TRASH, jnp.int32)
  src2 = jnp.concatenate([src_flat, pads]).reshape(nchunk + n0, ch)
  dst2 = jnp.concatenate([dst_flat, padd]).reshape(nchunk + n0, ch)
  zero = jnp.zeros((ch, D), jnp.float32)
  del ne
  p = _agg_cache[D](u, src2, dst2, zero)
  p0 = p[0, :N_NODES]
  if p.shape[0] == 2:
    p1 = p[1, :N_NODES]
  else:
    p1 = jnp.zeros_like(p0)
  return p0, p1


R = 1000
GRID = (N_NODES // R,)


def _row_spec(d):
  return pl.BlockSpec((R, d), lambda i: (i, 0))


def _full_spec(shape):
  return pl.BlockSpec(shape, lambda i: tuple(0 for _ in shape))


def _tc_pre_body(d0_ref, d1_ref, x_ref, dinv_ref, u1_ref):
  deg = d0_ref[...] + d1_ref[...] + 1.0
  dv = lax.rsqrt(deg)
  dinv_ref[...] = dv
  u1_ref[...] = dv * x_ref[...]


def _tc_pre(d0, d1, x):
  return pl.pallas_call(
      _tc_pre_body,
      grid=GRID,
      in_specs=[_row_spec(1), _row_spec(1), _row_spec(128)],
      out_specs=[_row_spec(1), _row_spec(128)],
      out_shape=[
          jax.ShapeDtypeStruct((N_NODES, 1), jnp.float32),
          jax.ShapeDtypeStruct((N_NODES, 128), jnp.float32),
      ],
  )(d0, d1, x)


def _tc1_body(p0, p1, u, dinv, W1, b1, W2, o):
  dv = dinv[...]
  a = dv * (p0[...] + p1[...] + u[...])
  h = jnp.maximum(jnp.dot(a, W1[...], preferred_element_type=jnp.float32)
                  + b1[...], 0.0)
  o[...] = dv * jnp.dot(h, W2[...], preferred_element_type=jnp.float32)


def _tc1(p0, p1, u, dinv, W1, b1, W2):
  return pl.pallas_call(
      _tc1_body,
      grid=GRID,
      in_specs=[_row_spec(128), _row_spec(128), _row_spec(128), _row_spec(1),
                _full_spec((128, 256)), _full_spec((1, 256)),
                _full_spec((256, 128))],
      out_specs=_row_spec(128),
      out_shape=jax.ShapeDtypeStruct((N_NODES, 128), jnp.float32),
  )(p0, p1, u, dinv, W1, b1, W2)


def _tc_mid_body(p0, p1, u, dinv, b, Wn, o):
  dv = dinv[...]
  h = jnp.maximum(dv * (p0[...] + p1[...] + u[...]) + b[...], 0.0)
  o[...] = dv * jnp.dot(h, Wn[...], preferred_element_type=jnp.float32)


def _tc_mid(p0, p1, u, dinv, b, Wn):
  d = u.shape[1]
  dn = Wn.shape[1]
  return pl.pallas_call(
      _tc_mid_body,
      grid=GRID,
      in_specs=[_row_spec(d), _row_spec(d), _row_spec(d), _row_spec(1),
                _full_spec((1, d)), _full_spec((d, dn))],
      out_specs=_row_spec(dn),
      out_shape=jax.ShapeDtypeStruct((N_NODES, dn), jnp.float32),
  )(p0, p1, u, dinv, b, Wn)


def _tc_h4_body(p0, p1, u, dinv, b, o):
  dv = dinv[...]
  h = jnp.maximum(dv * (p0[...] + p1[...] + u[...]) + b[...], 0.0)
  o[...] = dv * h


def _tc_h4(p0, p1, u, dinv, b):
  d = u.shape[1]
  return pl.pallas_call(
      _tc_h4_body,
      grid=GRID,
      in_specs=[_row_spec(d), _row_spec(d), _row_spec(d), _row_spec(1),
                _full_spec((1, d))],
      out_specs=_row_spec(d),
      out_shape=jax.ShapeDtypeStruct((N_NODES, d), jnp.float32),
  )(p0, p1, u, dinv, b)


def _tc_fin_body(p0, p1, u, dinv, Wm, bm, Wl, bl, mu, ls):
  a = dinv[...] * (p0[...] + p1[...] + u[...])
  mu[...] = jnp.dot(a, Wm[...], preferred_element_type=jnp.float32) + bm[...]
  ls[...] = jnp.dot(a, Wl[...], preferred_element_type=jnp.float32) + bl[...]


def _tc_fin(p0, p1, u, dinv, Wm, bm, Wl, bl):
  return pl.pallas_call(
      _tc_fin_body,
      grid=GRID,
      in_specs=[_row_spec(32), _row_spec(32), _row_spec(32), _row_spec(1),
                _full_spec((32, 16)), _full_spec((1, 16)),
                _full_spec((32, 16)), _full_spec((1, 16))],
      out_specs=[_row_spec(16), _row_spec(16)],
      out_shape=[
          jax.ShapeDtypeStruct((N_NODES, 16), jnp.float32),
          jax.ShapeDtypeStruct((N_NODES, 16), jnp.float32),
      ],
  )(p0, p1, u, dinv, Wm, bm, Wl, bl)


def kernel(x, edge_index, W1, b1, W2, b2, W3, b3, W4, b4,
           W_mu, b_mu, W_logstd, b_logstd):
  src = edge_index[0].astype(jnp.int32)
  dst = edge_index[1].astype(jnp.int32)

  b1r = b1.reshape(1, -1)
  b2r = b2.reshape(1, -1)
  b3r = b3.reshape(1, -1)
  b4r = b4.reshape(1, -1)
  bmr = b_mu.reshape(1, -1)
  blr = b_logstd.reshape(1, -1)

  ones = jnp.ones((N_NODES, 16), jnp.float32)
  g0, g1 = _agg(ones, src, dst, 16)
  dinv, u1 = _tc_pre(g0[:, :1], g1[:, :1], x)

  p0, p1 = _agg(u1, src, dst, 128)
  u2 = _tc1(p0, p1, u1, dinv, W1, b1r, W2)

  p0, p1 = _agg(u2, src, dst, 128)
  u3 = _tc_mid(p0, p1, u2, dinv, b2r, W3)

  p0, p1 = _agg(u3, src, dst, 64)
  u4 = _tc_mid(p0, p1, u3, dinv, b3r, W4)

  p0, p1 = _agg(u4, src, dst, 32)
  u5 = _tc_h4(p0, p1, u4, dinv, b4r)

  p0, p1 = _agg(u5, src, dst, 32)
  mu, logstd = _tc_fin(p0, p1, u5, dinv, W_mu, bmr, W_logstd, blr)
  return (mu, logstd)
